# baseline (speedup 1.0000x reference)
import functools

import jax
import jax.numpy as jnp
from jax import lax
from jax.experimental import pallas as pl
from jax.experimental.pallas import tpu as pltpu
from jax.experimental.pallas import tpu_sc as plsc

N = 10000
E = 320000
D_IN = 128
D_HID = 16
D_OUT = 40

NC = 2
NS = 16
NW = NC * NS

B = 80
CH = E // B // NW
N_ACC = 10240
RPT = N_ACC // NS
W = 5
NWAVES = CH // W


def _rsqrt16(d):
    i = plsc.bitcast(d, jnp.int32)
    i = jnp.int32(0x5F3759DF) - (i >> 1)
    y = plsc.bitcast(i, jnp.float32)
    for _ in range(3):
        y = y * (1.5 - 0.5 * d * y * y)
    return y


def _splat(vec_ref, j):
    return plsc.load_gather(vec_ref, [jnp.full((16,), 0, jnp.int32) + j])


def _edge_pass(src_v, dst_v, rows_v, g_src, acc_sh, sems):
    for k in range(W):
        pltpu.async_copy(g_src.at[src_v.at[k]], rows_v.at[k], sems[k])

    def group(gi, _):
        for k in range(W):
            j = gi * W + k
            pltpu.make_async_copy(g_src.at[src_v.at[j]], rows_v.at[k], sems[k]).wait()
            pltpu.sync_copy(rows_v.at[k], acc_sh.at[dst_v.at[j]], add=True)
            jn = j + W

            @pl.when(jn < CH)
            def _():
                pltpu.async_copy(g_src.at[src_v.at[jn]], rows_v.at[k], sems[k])
        return 0

    lax.fori_loop(0, NWAVES, group, 0)


@functools.lru_cache(maxsize=None)
def _sc_kernels():
    mesh = plsc.VectorSubcoreMesh(
        core_axis_name="c", subcore_axis_name="s", num_cores=NC, num_subcores=NS
    )

    @functools.partial(
        pl.kernel,
        out_type=[
            jax.ShapeDtypeStruct((NC, N_ACC, D_HID), jnp.float32),
            jax.ShapeDtypeStruct((NC, N_ACC, D_HID), jnp.float32),
            jax.ShapeDtypeStruct((N_ACC,), jnp.float32),
        ],
        mesh=mesh,
        scratch_types=[
            pltpu.VMEM((CH, B), jnp.int32),
            pltpu.VMEM((CH, B), jnp.int32),
            pltpu.VMEM((2 * CH, B), jnp.int32),
            pltpu.VMEM((W, B, D_HID), jnp.float32),
            pltpu.VMEM((RPT, D_HID), jnp.float32),
            pltpu.VMEM((B,), jnp.float32),
            pltpu.VMEM((RPT,), jnp.float32),
            pltpu.VMEM((RPT,), jnp.float32),
            pltpu.VMEM_SHARED((N_ACC, D_HID), jnp.float32),
            pltpu.VMEM_SHARED((N_ACC,), jnp.float32),
            pltpu.SemaphoreType.DMA,
        ]
        + [pltpu.SemaphoreType.DMA] * W,
        compiler_params=pltpu.CompilerParams(use_tc_tiling_on_sc=False, needs_layout_passes=False),
    )
    def agg1_kernel(
        src_hbm, dst_hbm, h1_hbm, zero_hbm, zero1_hbm,
        q_hbm, g1_hbm, dinv_hbm,
        src_v, dst_v, dst2_v, rows_v, slab_v, ones_v, deg_v, dinv_v, acc_sh, deg_sh,
        sem, *sems,
    ):
        cid = lax.axis_index("c")
        sid = lax.axis_index("s")
        wid = sid * NC + cid
        base = sid * RPT

        @pl.when(sid == 0)
        def _():
            pltpu.sync_copy(zero_hbm, acc_sh)
            pltpu.sync_copy(zero1_hbm, deg_sh)

        for i in range(B // 16):
            ones_v[pl.ds(i * 16, 16)] = jnp.full((16,), 1.0, jnp.float32)
        pltpu.sync_copy(src_hbm.at[wid], src_v)
        pltpu.sync_copy(dst_hbm.at[wid], dst_v)
        pltpu.sync_copy(dst_hbm.at[2 * sid], dst2_v.at[pl.ds(0, CH)])
        pltpu.sync_copy(dst_hbm.at[2 * sid + 1], dst2_v.at[pl.ds(CH, CH)])
        pltpu.sync_copy(h1_hbm.at[pl.ds(base, RPT)], slab_v)
        plsc.subcore_barrier()

        def dbody(j, _):
            pltpu.async_copy(ones_v, deg_sh.at[dst2_v.at[j]], sem, add=True)
            return 0

        lax.fori_loop(0, 2 * CH, dbody, 0)

        def ddrain(j, _):
            pltpu.make_async_copy(ones_v, deg_sh.at[dst2_v.at[j]], sem).wait()
            return 0

        lax.fori_loop(0, 2 * CH, ddrain, 0)
        plsc.subcore_barrier()

        pltpu.sync_copy(deg_sh.at[pl.ds(base, RPT)], deg_v)

        def dinv_blk(i, _):
            d = deg_v[pl.ds(i * 16, 16)] + 1.0
            dinv_v[pl.ds(i * 16, 16)] = _rsqrt16(d)
            return 0

        lax.fori_loop(0, RPT // 16, dinv_blk, 0)

        def scale_row(j, _):
            slab_v[j] = slab_v[j] * _splat(dinv_v, j)
            return 0

        lax.fori_loop(0, RPT, scale_row, 0)

        pltpu.sync_copy(slab_v, g1_hbm.at[cid, pl.ds(base, RPT)])

        @pl.when(cid == 0)
        def _():
            pltpu.sync_copy(dinv_v, dinv_hbm.at[pl.ds(base, RPT)])

        plsc.subcore_barrier()
        _edge_pass(src_v, dst_v, rows_v, g1_hbm.at[cid], acc_sh, sems)
        plsc.subcore_barrier()

        pltpu.sync_copy(acc_sh.at[pl.ds(base, RPT)], slab_v)
        pltpu.sync_copy(slab_v, q_hbm.at[cid, pl.ds(base, RPT)])

    @functools.partial(
        pl.kernel,
        out_type=[
            jax.ShapeDtypeStruct((NC, N_ACC, D_HID), jnp.float32),
            jax.ShapeDtypeStruct((NC, N_ACC, D_HID), jnp.float32),
        ],
        mesh=mesh,
        scratch_types=[
            pltpu.VMEM((CH, B), jnp.int32),
            pltpu.VMEM((CH, B), jnp.int32),
            pltpu.VMEM((W, B, D_HID), jnp.float32),
            pltpu.VMEM((RPT, D_HID), jnp.float32),
            pltpu.VMEM((RPT, D_HID), jnp.float32),
            pltpu.VMEM((RPT, D_HID), jnp.float32),
            pltpu.VMEM((RPT,), jnp.float32),
            pltpu.VMEM((16,), jnp.float32),
            pltpu.VMEM_SHARED((N_ACC, D_HID), jnp.float32),
        ]
        + [pltpu.SemaphoreType.DMA] * W,
        compiler_params=pltpu.CompilerParams(use_tc_tiling_on_sc=False, needs_layout_passes=False),
    )
    def agg2_kernel(
        src_hbm, dst_hbm, q_hbm, g1_hbm, dinv_hbm, b1_hbm, zero_hbm,
        r_hbm, g2_hbm,
        src_v, dst_v, rows_v, slab_v, q0_v, q1_v, dinv_v, b1_v, acc_sh, *sems,
    ):
        cid = lax.axis_index("c")
        sid = lax.axis_index("s")
        wid = sid * NC + cid
        base = sid * RPT

        @pl.when(sid == 0)
        def _():
            pltpu.sync_copy(zero_hbm, acc_sh)

        pltpu.sync_copy(src_hbm.at[wid], src_v)
        pltpu.sync_copy(dst_hbm.at[wid], dst_v)
        pltpu.sync_copy(q_hbm.at[0, pl.ds(base, RPT)], q0_v)
        pltpu.sync_copy(q_hbm.at[1, pl.ds(base, RPT)], q1_v)
        pltpu.sync_copy(g1_hbm.at[cid, pl.ds(base, RPT)], slab_v)
        pltpu.sync_copy(dinv_hbm.at[pl.ds(base, RPT)], dinv_v)
        pltpu.sync_copy(b1_hbm, b1_v)
        bias = b1_v[...]

        def relu_row(j, _):
            d = _splat(dinv_v, j)
            s = (q0_v[j] + q1_v[j] + slab_v[j]) * d + bias
            slab_v[j] = jnp.maximum(s, 0.0) * d
            return 0

        lax.fori_loop(0, RPT, relu_row, 0)

        pltpu.sync_copy(slab_v, g2_hbm.at[cid, pl.ds(base, RPT)])
        plsc.subcore_barrier()
        _edge_pass(src_v, dst_v, rows_v, g2_hbm.at[cid], acc_sh, sems)
        plsc.subcore_barrier()

        pltpu.sync_copy(acc_sh.at[pl.ds(base, RPT)], slab_v)
        pltpu.sync_copy(slab_v, r_hbm.at[cid, pl.ds(base, RPT)])

    return agg1_kernel, agg2_kernel


def _mm1_body(x_ref, w_ref, h_ref):
    h_ref[...] = jnp.dot(x_ref[...], w_ref[...], preferred_element_type=jnp.float32)


def _mm2_body(r_ref, g2_ref, dinv_ref, w_ref, b_ref, out_ref):
    a = (r_ref[0, :N, :] + r_ref[1, :N, :] + g2_ref[0, :N, :]) * dinv_ref[...][:N, None]
    out_ref[...] = (
        jnp.dot(a, w_ref[...], preferred_element_type=jnp.float32)
        + b_ref[...][None, :]
    )


def kernel(x, edge_index, W1, b1, W2, b2):
    src = edge_index[0].astype(jnp.int32).reshape(NW, CH, B)
    dst = edge_index[1].astype(jnp.int32).reshape(NW, CH, B)
    x_pad = jnp.pad(x, ((0, N_ACC - N), (0, 0)))
    z1 = jnp.zeros((N_ACC,), jnp.float32)
    z16 = jnp.zeros((N_ACC, D_HID), jnp.float32)
    _agg1_kernel, _agg2_kernel = _sc_kernels()

    h1 = pl.pallas_call(
        _mm1_body,
        out_shape=jax.ShapeDtypeStruct((N_ACC, D_HID), jnp.float32),
    )(x_pad, W1)

    q, g1, dinv = _agg1_kernel(src, dst, h1, z16, z1)
    r, g2 = _agg2_kernel(src, dst, q, g1, dinv, b1, z16)

    out = pl.pallas_call(
        _mm2_body,
        out_shape=jax.ShapeDtypeStruct((N, D_OUT), jnp.float32),
    )(r, g2, dinv, W2, b2)
    return out


# speedup vs baseline: 1.0164x; 1.0164x over previous
import functools

import jax
import jax.numpy as jnp
from jax import lax
from jax.experimental import pallas as pl
from jax.experimental.pallas import tpu as pltpu
from jax.experimental.pallas import tpu_sc as plsc

N = 10000
E = 320000
D_IN = 128
D_HID = 16
D_OUT = 40

NC = 2
NS = 16
NW = NC * NS

B = 80
CH = E // B // NW
N_ACC = 10240
RPT = N_ACC // NS
W = 5
NWAVES = CH // W


def _rsqrt16(d):
    i = plsc.bitcast(d, jnp.int32)
    i = jnp.int32(0x5F3759DF) - (i >> 1)
    y = plsc.bitcast(i, jnp.float32)
    for _ in range(3):
        y = y * (1.5 - 0.5 * d * y * y)
    return y


def _splat(vec_ref, j):
    return plsc.load_gather(vec_ref, [jnp.full((16,), 0, jnp.int32) + j])


def _edge_pass(src_v, dst_v, rows_v, g_src, acc_sh, sems):
    for k in range(W):
        pltpu.async_copy(g_src.at[src_v.at[pl.ds(k * B, B)]], rows_v.at[k], sems[k])

    def group(gi, _):
        for k in range(W):
            j = gi * W + k
            pltpu.make_async_copy(
                g_src.at[src_v.at[pl.ds(j * B, B)]], rows_v.at[k], sems[k]
            ).wait()
            pltpu.sync_copy(rows_v.at[k], acc_sh.at[dst_v.at[j]], add=True)
            jn = j + W

            @pl.when(jn < CH)
            def _():
                pltpu.async_copy(g_src.at[src_v.at[pl.ds(jn * B, B)]], rows_v.at[k], sems[k])
        return 0

    lax.fori_loop(0, NWAVES, group, 0)


@functools.lru_cache(maxsize=None)
def _sc_kernels():
    mesh = plsc.VectorSubcoreMesh(
        core_axis_name="c", subcore_axis_name="s", num_cores=NC, num_subcores=NS
    )

    @functools.partial(
        pl.kernel,
        out_type=[
            jax.ShapeDtypeStruct((NC, N_ACC, D_HID), jnp.float32),
            jax.ShapeDtypeStruct((NC, N_ACC, D_HID), jnp.float32),
            jax.ShapeDtypeStruct((N_ACC,), jnp.float32),
        ],
        mesh=mesh,
        scratch_types=[
            pltpu.VMEM((CH * B,), jnp.int32),
            pltpu.VMEM((CH, B), jnp.int32),
            pltpu.VMEM((2 * CH, B), jnp.int32),
            pltpu.VMEM((W, B, D_HID), jnp.float32),
            pltpu.VMEM((RPT, D_HID), jnp.float32),
            pltpu.VMEM((B,), jnp.float32),
            pltpu.VMEM((RPT,), jnp.float32),
            pltpu.VMEM((RPT,), jnp.float32),
            pltpu.VMEM_SHARED((N_ACC, D_HID), jnp.float32),
            pltpu.VMEM_SHARED((N_ACC,), jnp.float32),
            pltpu.SemaphoreType.DMA,
        ]
        + [pltpu.SemaphoreType.DMA] * W,
        compiler_params=pltpu.CompilerParams(use_tc_tiling_on_sc=False, needs_layout_passes=False),
    )
    def agg1_kernel(
        src_hbm, dst_hbm, h1_hbm, zero_hbm, zero1_hbm,
        q_hbm, g1_hbm, dinv_hbm,
        src_v, dst_v, dst2_v, rows_v, slab_v, ones_v, deg_v, dinv_v, acc_sh, deg_sh,
        sem, *sems,
    ):
        cid = lax.axis_index("c")
        sid = lax.axis_index("s")
        wid = sid * NC + cid
        base = sid * RPT

        @pl.when(sid == 0)
        def _():
            pltpu.sync_copy(zero_hbm, acc_sh)
            pltpu.sync_copy(zero1_hbm, deg_sh)

        for i in range(B // 16):
            ones_v[pl.ds(i * 16, 16)] = jnp.full((16,), 1.0, jnp.float32)
        pltpu.sync_copy(src_hbm.at[pl.ds(wid * CH * B, CH * B)], src_v)
        pltpu.sync_copy(dst_hbm.at[wid], dst_v)
        pltpu.sync_copy(dst_hbm.at[2 * sid], dst2_v.at[pl.ds(0, CH)])
        pltpu.sync_copy(dst_hbm.at[2 * sid + 1], dst2_v.at[pl.ds(CH, CH)])
        pltpu.sync_copy(h1_hbm.at[pl.ds(base, RPT)], slab_v)
        plsc.subcore_barrier()

        def dbody(j, _):
            pltpu.async_copy(ones_v, deg_sh.at[dst2_v.at[j]], sem, add=True)
            return 0

        lax.fori_loop(0, 2 * CH, dbody, 0)

        def ddrain(j, _):
            pltpu.make_async_copy(ones_v, deg_sh.at[dst2_v.at[j]], sem).wait()
            return 0

        lax.fori_loop(0, 2 * CH, ddrain, 0)
        plsc.subcore_barrier()

        pltpu.sync_copy(deg_sh.at[pl.ds(base, RPT)], deg_v)

        def dinv_blk(i, _):
            d = deg_v[pl.ds(i * 16, 16)] + 1.0
            dinv_v[pl.ds(i * 16, 16)] = _rsqrt16(d)
            return 0

        lax.fori_loop(0, RPT // 16, dinv_blk, 0)

        def scale_row(j, _):
            slab_v[j] = slab_v[j] * _splat(dinv_v, j)
            return 0

        lax.fori_loop(0, RPT, scale_row, 0)

        pltpu.sync_copy(slab_v, g1_hbm.at[cid, pl.ds(base, RPT)])

        @pl.when(cid == 0)
        def _():
            pltpu.sync_copy(dinv_v, dinv_hbm.at[pl.ds(base, RPT)])

        plsc.subcore_barrier()
        _edge_pass(src_v, dst_v, rows_v, g1_hbm.at[cid], acc_sh, sems)
        plsc.subcore_barrier()

        pltpu.sync_copy(acc_sh.at[pl.ds(base, RPT)], slab_v)
        pltpu.sync_copy(slab_v, q_hbm.at[cid, pl.ds(base, RPT)])

    @functools.partial(
        pl.kernel,
        out_type=[
            jax.ShapeDtypeStruct((NC, N_ACC, D_HID), jnp.float32),
            jax.ShapeDtypeStruct((NC, N_ACC, D_HID), jnp.float32),
        ],
        mesh=mesh,
        scratch_types=[
            pltpu.VMEM((CH * B,), jnp.int32),
            pltpu.VMEM((CH, B), jnp.int32),
            pltpu.VMEM((W, B, D_HID), jnp.float32),
            pltpu.VMEM((RPT, D_HID), jnp.float32),
            pltpu.VMEM((RPT, D_HID), jnp.float32),
            pltpu.VMEM((RPT, D_HID), jnp.float32),
            pltpu.VMEM((RPT,), jnp.float32),
            pltpu.VMEM((16,), jnp.float32),
            pltpu.VMEM_SHARED((N_ACC, D_HID), jnp.float32),
        ]
        + [pltpu.SemaphoreType.DMA] * W,
        compiler_params=pltpu.CompilerParams(use_tc_tiling_on_sc=False, needs_layout_passes=False),
    )
    def agg2_kernel(
        src_hbm, dst_hbm, q_hbm, g1_hbm, dinv_hbm, b1_hbm, zero_hbm,
        r_hbm, g2_hbm,
        src_v, dst_v, rows_v, slab_v, q0_v, q1_v, dinv_v, b1_v, acc_sh, *sems,
    ):
        cid = lax.axis_index("c")
        sid = lax.axis_index("s")
        wid = sid * NC + cid
        base = sid * RPT

        @pl.when(sid == 0)
        def _():
            pltpu.sync_copy(zero_hbm, acc_sh)

        pltpu.sync_copy(src_hbm.at[pl.ds(wid * CH * B, CH * B)], src_v)
        pltpu.sync_copy(dst_hbm.at[wid], dst_v)
        pltpu.sync_copy(q_hbm.at[0, pl.ds(base, RPT)], q0_v)
        pltpu.sync_copy(q_hbm.at[1, pl.ds(base, RPT)], q1_v)
        pltpu.sync_copy(g1_hbm.at[cid, pl.ds(base, RPT)], slab_v)
        pltpu.sync_copy(dinv_hbm.at[pl.ds(base, RPT)], dinv_v)
        pltpu.sync_copy(b1_hbm, b1_v)
        bias = b1_v[...]

        def relu_row(j, _):
            d = _splat(dinv_v, j)
            s = (q0_v[j] + q1_v[j] + slab_v[j]) * d + bias
            slab_v[j] = jnp.maximum(s, 0.0) * d
            return 0

        lax.fori_loop(0, RPT, relu_row, 0)

        pltpu.sync_copy(slab_v, g2_hbm.at[cid, pl.ds(base, RPT)])
        plsc.subcore_barrier()
        _edge_pass(src_v, dst_v, rows_v, g2_hbm.at[cid], acc_sh, sems)
        plsc.subcore_barrier()

        pltpu.sync_copy(acc_sh.at[pl.ds(base, RPT)], slab_v)
        pltpu.sync_copy(slab_v, r_hbm.at[cid, pl.ds(base, RPT)])

    return agg1_kernel, agg2_kernel


def _mm1_body(x_ref, w_ref, h_ref):
    h_ref[:N, :] = jnp.dot(x_ref[...], w_ref[...], preferred_element_type=jnp.float32)
    h_ref[N:, :] = jnp.zeros((N_ACC - N, D_HID), jnp.float32)


def _mm2_body(r_ref, g2_ref, dinv_ref, w_ref, b_ref, out_ref):
    a = (r_ref[0, :N, :] + r_ref[1, :N, :] + g2_ref[0, :N, :]) * dinv_ref[...][:N, None]
    out_ref[...] = (
        jnp.dot(a, w_ref[...], preferred_element_type=jnp.float32)
        + b_ref[...][None, :]
    )


def kernel(x, edge_index, W1, b1, W2, b2):
    src = edge_index[0].astype(jnp.int32)
    dst = edge_index[1].astype(jnp.int32).reshape(NW, CH, B)
    z1 = jnp.zeros((N_ACC,), jnp.float32)
    z16 = jnp.zeros((N_ACC, D_HID), jnp.float32)
    _agg1_kernel, _agg2_kernel = _sc_kernels()

    h1 = pl.pallas_call(
        _mm1_body,
        out_shape=jax.ShapeDtypeStruct((N_ACC, D_HID), jnp.float32),
    )(x, W1)

    q, g1, dinv = _agg1_kernel(src, dst, h1, z16, z1)
    r, g2 = _agg2_kernel(src, dst, q, g1, dinv, b1, z16)

    out = pl.pallas_call(
        _mm2_body,
        out_shape=jax.ShapeDtypeStruct((N, D_OUT), jnp.float32),
    )(r, g2, dinv, W2, b2)
    return out


# speedup vs baseline: 1.0457x; 1.0288x over previous
import functools

import jax
import jax.numpy as jnp
from jax import lax
from jax.experimental import pallas as pl
from jax.experimental.pallas import tpu as pltpu
from jax.experimental.pallas import tpu_sc as plsc

N = 10000
E = 320000
D_IN = 128
D_HID = 16
D_OUT = 40

NC = 2
NS = 16
NW = NC * NS

B = 80
CH = E // B // NW
N_ACC = 10240
RPT = N_ACC // NS
W = 5
NWAVES = CH // W


def _rsqrt16(d):
    i = plsc.bitcast(d, jnp.int32)
    i = jnp.int32(0x5F3759DF) - (i >> 1)
    y = plsc.bitcast(i, jnp.float32)
    for _ in range(3):
        y = y * (1.5 - 0.5 * d * y * y)
    return y


def _splat(vec_ref, j):
    return plsc.load_gather(vec_ref, [jnp.full((16,), 0, jnp.int32) + j])


def _edge_pass(src_v, dst_v, rows_v, g_src, acc_sh, sems):
    for k in range(W):
        pltpu.async_copy(g_src.at[src_v.at[pl.ds(k * B, B)]], rows_v.at[k], sems[k])

    def group(gi, _):
        for k in range(W):
            j = gi * W + k
            pltpu.make_async_copy(
                g_src.at[src_v.at[pl.ds(j * B, B)]], rows_v.at[k], sems[k]
            ).wait()
            pltpu.sync_copy(rows_v.at[k], acc_sh.at[dst_v.at[j]], add=True)
            jn = j + W

            @pl.when(jn < CH)
            def _():
                pltpu.async_copy(g_src.at[src_v.at[pl.ds(jn * B, B)]], rows_v.at[k], sems[k])
        return 0

    lax.fori_loop(0, NWAVES, group, 0)


@functools.lru_cache(maxsize=None)
def _sc_kernels():
    mesh = plsc.VectorSubcoreMesh(
        core_axis_name="c", subcore_axis_name="s", num_cores=NC, num_subcores=NS
    )

    @functools.partial(
        pl.kernel,
        out_type=[
            jax.ShapeDtypeStruct((N_ACC,), jnp.float32),
            jax.ShapeDtypeStruct((N_ACC,), jnp.float32),
        ],
        mesh=mesh,
        scratch_types=[
            pltpu.VMEM((CH, B), jnp.int32),
            pltpu.VMEM((B,), jnp.float32),
            pltpu.VMEM((RPT,), jnp.float32),
            pltpu.VMEM_SHARED((N_ACC,), jnp.float32),
            pltpu.SemaphoreType.DMA,
        ],
    )
    def deg_kernel(dst_hbm, zero_hbm, out0_hbm, out1_hbm, dst_v, ones_v, slab_v, acc_sh, sem):
        cid = lax.axis_index("c")
        sid = lax.axis_index("s")
        wid = sid * NC + cid

        @pl.when(sid == 0)
        def _():
            pltpu.sync_copy(zero_hbm, acc_sh)

        for i in range(B // 16):
            ones_v[pl.ds(i * 16, 16)] = jnp.full((16,), 1.0, jnp.float32)
        pltpu.sync_copy(dst_hbm.at[wid], dst_v)
        plsc.subcore_barrier()

        def body(j, _):
            pltpu.async_copy(ones_v, acc_sh.at[dst_v.at[j]], sem, add=True)
            return 0

        lax.fori_loop(0, CH, body, 0)

        def drain(j, _):
            pltpu.make_async_copy(ones_v, acc_sh.at[dst_v.at[j]], sem).wait()
            return 0

        lax.fori_loop(0, CH, drain, 0)
        plsc.subcore_barrier()

        pltpu.sync_copy(acc_sh.at[pl.ds(sid * RPT, RPT)], slab_v)

        @pl.when(cid == 0)
        def _():
            pltpu.sync_copy(slab_v, out0_hbm.at[pl.ds(sid * RPT, RPT)])

        @pl.when(cid == 1)
        def _():
            pltpu.sync_copy(slab_v, out1_hbm.at[pl.ds(sid * RPT, RPT)])

    @functools.partial(
        pl.kernel,
        out_type=[
            jax.ShapeDtypeStruct((NC, N_ACC, D_HID), jnp.float32),
            jax.ShapeDtypeStruct((NC, N_ACC, D_HID), jnp.float32),
            jax.ShapeDtypeStruct((N_ACC,), jnp.float32),
        ],
        mesh=mesh,
        scratch_types=[
            pltpu.VMEM((CH * B,), jnp.int32),
            pltpu.VMEM((CH, B), jnp.int32),
            pltpu.VMEM((W, B, D_HID), jnp.float32),
            pltpu.VMEM((RPT, D_HID), jnp.float32),
            pltpu.VMEM((RPT,), jnp.float32),
            pltpu.VMEM((RPT,), jnp.float32),
            pltpu.VMEM((RPT,), jnp.float32),
            pltpu.VMEM_SHARED((N_ACC, D_HID), jnp.float32),
        ]
        + [pltpu.SemaphoreType.DMA] * W,
        compiler_params=pltpu.CompilerParams(use_tc_tiling_on_sc=False, needs_layout_passes=False),
    )
    def agg1_kernel(
        src_hbm, dst_hbm, h1_hbm, p0_hbm, p1_hbm, zero_hbm,
        q_hbm, g1_hbm, dinv_hbm,
        src_v, dst_v, rows_v, slab_v, p0_v, p1_v, dinv_v, acc_sh, *sems,
    ):
        cid = lax.axis_index("c")
        sid = lax.axis_index("s")
        wid = sid * NC + cid
        base = sid * RPT

        @pl.when(sid == 0)
        def _():
            pltpu.sync_copy(zero_hbm, acc_sh)

        pltpu.sync_copy(src_hbm.at[pl.ds(wid * CH * B, CH * B)], src_v)
        pltpu.sync_copy(dst_hbm.at[wid], dst_v)
        pltpu.sync_copy(p0_hbm.at[pl.ds(base, RPT)], p0_v)
        pltpu.sync_copy(p1_hbm.at[pl.ds(base, RPT)], p1_v)
        pltpu.sync_copy(h1_hbm.at[pl.ds(base, RPT)], slab_v)

        def dinv_blk(i, _):
            d = p0_v[pl.ds(i * 16, 16)] + p1_v[pl.ds(i * 16, 16)] + 1.0
            dinv_v[pl.ds(i * 16, 16)] = _rsqrt16(d)
            return 0

        lax.fori_loop(0, RPT // 16, dinv_blk, 0)

        def scale_row(j, _):
            slab_v[j] = slab_v[j] * _splat(dinv_v, j)
            return 0

        lax.fori_loop(0, RPT, scale_row, 0)

        pltpu.sync_copy(slab_v, g1_hbm.at[cid, pl.ds(base, RPT)])

        @pl.when(cid == 0)
        def _():
            pltpu.sync_copy(dinv_v, dinv_hbm.at[pl.ds(base, RPT)])

        plsc.subcore_barrier()
        _edge_pass(src_v, dst_v, rows_v, g1_hbm.at[cid], acc_sh, sems)
        plsc.subcore_barrier()

        pltpu.sync_copy(acc_sh.at[pl.ds(base, RPT)], slab_v)
        pltpu.sync_copy(slab_v, q_hbm.at[cid, pl.ds(base, RPT)])

    @functools.partial(
        pl.kernel,
        out_type=[
            jax.ShapeDtypeStruct((NC, N_ACC, D_HID), jnp.float32),
            jax.ShapeDtypeStruct((NC, N_ACC, D_HID), jnp.float32),
        ],
        mesh=mesh,
        scratch_types=[
            pltpu.VMEM((CH * B,), jnp.int32),
            pltpu.VMEM((CH, B), jnp.int32),
            pltpu.VMEM((W, B, D_HID), jnp.float32),
            pltpu.VMEM((RPT, D_HID), jnp.float32),
            pltpu.VMEM((RPT, D_HID), jnp.float32),
            pltpu.VMEM((RPT, D_HID), jnp.float32),
            pltpu.VMEM((RPT,), jnp.float32),
            pltpu.VMEM((16,), jnp.float32),
            pltpu.VMEM_SHARED((N_ACC, D_HID), jnp.float32),
        ]
        + [pltpu.SemaphoreType.DMA] * W,
        compiler_params=pltpu.CompilerParams(use_tc_tiling_on_sc=False, needs_layout_passes=False),
    )
    def agg2_kernel(
        src_hbm, dst_hbm, q_hbm, g1_hbm, dinv_hbm, b1_hbm, zero_hbm,
        r_hbm, g2_hbm,
        src_v, dst_v, rows_v, slab_v, q0_v, q1_v, dinv_v, b1_v, acc_sh, *sems,
    ):
        cid = lax.axis_index("c")
        sid = lax.axis_index("s")
        wid = sid * NC + cid
        base = sid * RPT

        @pl.when(sid == 0)
        def _():
            pltpu.sync_copy(zero_hbm, acc_sh)

        pltpu.sync_copy(src_hbm.at[pl.ds(wid * CH * B, CH * B)], src_v)
        pltpu.sync_copy(dst_hbm.at[wid], dst_v)
        pltpu.sync_copy(q_hbm.at[0, pl.ds(base, RPT)], q0_v)
        pltpu.sync_copy(q_hbm.at[1, pl.ds(base, RPT)], q1_v)
        pltpu.sync_copy(g1_hbm.at[cid, pl.ds(base, RPT)], slab_v)
        pltpu.sync_copy(dinv_hbm.at[pl.ds(base, RPT)], dinv_v)
        pltpu.sync_copy(b1_hbm, b1_v)
        bias = b1_v[...]

        def relu_row(j, _):
            d = _splat(dinv_v, j)
            s = (q0_v[j] + q1_v[j] + slab_v[j]) * d + bias
            slab_v[j] = jnp.maximum(s, 0.0) * d
            return 0

        lax.fori_loop(0, RPT, relu_row, 0)

        pltpu.sync_copy(slab_v, g2_hbm.at[cid, pl.ds(base, RPT)])
        plsc.subcore_barrier()
        _edge_pass(src_v, dst_v, rows_v, g2_hbm.at[cid], acc_sh, sems)
        plsc.subcore_barrier()

        pltpu.sync_copy(acc_sh.at[pl.ds(base, RPT)], slab_v)
        pltpu.sync_copy(slab_v, r_hbm.at[cid, pl.ds(base, RPT)])

    return deg_kernel, agg1_kernel, agg2_kernel


def _mm1_body(x_ref, w_ref, h_ref):
    h_ref[:N, :] = jnp.dot(x_ref[...], w_ref[...], preferred_element_type=jnp.float32)
    h_ref[N:, :] = jnp.zeros((N_ACC - N, D_HID), jnp.float32)


def _mm2_body(r_ref, g2_ref, dinv_ref, w_ref, b_ref, out_ref):
    a = (r_ref[0, :N, :] + r_ref[1, :N, :] + g2_ref[0, :N, :]) * dinv_ref[...][:N, None]
    out_ref[...] = (
        jnp.dot(a, w_ref[...], preferred_element_type=jnp.float32)
        + b_ref[...][None, :]
    )


def kernel(x, edge_index, W1, b1, W2, b2):
    src = edge_index[0].astype(jnp.int32)
    dst = edge_index[1].astype(jnp.int32).reshape(NW, CH, B)
    z1 = jnp.zeros((N_ACC,), jnp.float32)
    z16 = jnp.zeros((N_ACC, D_HID), jnp.float32)
    _deg_kernel, _agg1_kernel, _agg2_kernel = _sc_kernels()

    h1 = pl.pallas_call(
        _mm1_body,
        out_shape=jax.ShapeDtypeStruct((N_ACC, D_HID), jnp.float32),
    )(x, W1)

    p0, p1 = _deg_kernel(dst, z1)
    q, g1, dinv = _agg1_kernel(src, dst, h1, p0, p1, z16)
    r, g2 = _agg2_kernel(src, dst, q, g1, dinv, b1, z16)

    out = pl.pallas_call(
        _mm2_body,
        out_shape=jax.ShapeDtypeStruct((N, D_OUT), jnp.float32),
    )(r, g2, dinv, W2, b2)
    return out
